# baseline (speedup 1.0000x reference)
import functools

import jax
import jax.numpy as jnp
from jax import lax
from jax.experimental import pallas as pl
from jax.experimental.pallas import tpu as pltpu
from jax.experimental.pallas import tpu_sc as plsc

N = 100000
F = 9
V = 100
H = 128
L = 16
C = 128
NW = 32
NCHUNK = (N + C - 1) // C


def _sc_body(x_hbm, table_hbm, out_hbm, raw0, raw1, idx0, idx1, acc0, acc1,
             idx_sem, gather_sem, out_sem0, out_sem1):
    raws = (raw0, raw1)
    idxs = (idx0, idx1)
    accs = (acc0, acc1)
    out_sems = (out_sem0, out_sem1)

    cid = lax.axis_index("c")
    sid = lax.axis_index("s")
    wid = sid * 2 + cid
    nj = (NCHUNK - wid + NW - 1) // NW

    lanes9 = lax.broadcasted_iota(jnp.int32, (L,), 0) * 9
    zeros = jnp.zeros((L,), jnp.float32)

    def chunk_base(j):
        chunk = wid + j * NW
        return jnp.minimum(chunk * C, N - C)

    def fire_idx(j, b):
        return pltpu.async_copy(x_hbm.at[pl.ds(chunk_base(j) * F, C * F)],
                                raws[b], idx_sem)

    def do_chunk(j, b):
        pltpu.make_async_copy(x_hbm.at[pl.ds(0, C * F)], raws[b],
                              idx_sem).wait()

        @pl.when(j + 1 < nj)
        def _():
            fire_idx(j + 1, 1 - b)

        for g in range(C // L):
            rows9 = g * (L * 9) + lanes9
            for i in range(F):
                vals = plsc.load_gather(raws[b], [rows9 + i])
                idxs[b][i, pl.ds(g * L, L)] = vals + (i * V)

        @pl.when(j >= 2)
        def _():
            pltpu.make_async_copy(accs[b], out_hbm.at[pl.ds(0, C)],
                                  out_sems[b]).wait()

        def zero_row(r, carry):
            for cc in range(H // L):
                accs[b][r, pl.ds(cc * L, L)] = zeros
            return carry

        lax.fori_loop(0, C, zero_row, 0)

        cps = [
            pltpu.async_copy(table_hbm.at[idxs[b].at[i]], accs[b],
                             gather_sem, add=True)
            for i in range(F)
        ]
        for cp in cps:
            cp.wait()

        pltpu.async_copy(accs[b], out_hbm.at[pl.ds(chunk_base(j), C)],
                         out_sems[b])

    @pl.when(nj > 0)
    def _():
        fire_idx(0, 0)

    def pair_step(jj, carry):
        j0 = jj * 2

        @pl.when(j0 < nj)
        def _():
            do_chunk(j0, 0)

        @pl.when(j0 + 1 < nj)
        def _():
            do_chunk(j0 + 1, 1)

        return carry

    lax.fori_loop(0, (nj + 1) // 2, pair_step, 0)

    for b in range(2):
        @pl.when(nj >= 2 - b)
        def _():
            pltpu.make_async_copy(accs[b], out_hbm.at[pl.ds(0, C)],
                                  out_sems[b]).wait()


@functools.lru_cache(maxsize=1)
def _build_encoder():
    @functools.partial(
        pl.kernel,
        out_type=jax.ShapeDtypeStruct((N, H), jnp.float32),
        mesh=plsc.VectorSubcoreMesh(core_axis_name="c", subcore_axis_name="s"),
        compiler_params=pltpu.CompilerParams(needs_layout_passes=False),
        scratch_types=[
            pltpu.VMEM((C * F,), jnp.int32),
            pltpu.VMEM((C * F,), jnp.int32),
            pltpu.VMEM((F, C), jnp.int32),
            pltpu.VMEM((F, C), jnp.int32),
            pltpu.VMEM((C, H), jnp.float32),
            pltpu.VMEM((C, H), jnp.float32),
            pltpu.SemaphoreType.DMA,
            pltpu.SemaphoreType.DMA,
            pltpu.SemaphoreType.DMA,
            pltpu.SemaphoreType.DMA,
        ],
    )
    def _sc_encoder(x_hbm, table_hbm, out_hbm, raw0, raw1, idx0, idx1,
                    acc0, acc1, idx_sem, gather_sem, out_sem0, out_sem1):
        _sc_body(x_hbm, table_hbm, out_hbm, raw0, raw1, idx0, idx1,
                 acc0, acc1, idx_sem, gather_sem, out_sem0, out_sem1)

    return _sc_encoder


def kernel(x, emb):
    table = emb.reshape(F * V, H)
    return _build_encoder()(x.astype(jnp.int32).reshape(N * F), table)
